# baseline (device time: 882893 ns/iter reference)
import jax
import jax.numpy as jnp
from jax import lax
from jax.experimental import pallas as pl
from jax.experimental.pallas import tpu as pltpu

N_DEV = 32


def kernel(x, w_mat, scale_x, scale_w):
    m, k = x.shape
    _, n = w_mat.shape
    ch = m // N_DEV
    ha = n // 2

    def body(x_ref, w_ref, sx_ref, sw_ref, out_ref, comm_a, comm_b,
             sa_send, sa_recv, sb_send, sb_recv,
             ga_send, ga_recv, gb_send, gb_recv,
             cr_a, cr_b, cg_a, cg_b):
        me = lax.axis_index("i")
        left = (me - 1) % N_DEV
        right = (me + 1) % N_DEV

        out_ref[...] = jnp.dot(
            x_ref[...].astype(jnp.bfloat16),
            w_ref[...].astype(jnp.bfloat16),
            preferred_element_type=jnp.float32,
        )

        barrier_sem = pltpu.get_barrier_semaphore()
        for nbr in (left, right):
            pl.semaphore_signal(
                barrier_sem, inc=1,
                device_id=(nbr,), device_id_type=pl.DeviceIdType.MESH,
            )
        pl.semaphore_wait(barrier_sem, 2)

        last_a = {}
        last_b = {}
        for s in range(N_DEV - 1):
            slot = s % 2
            ia_s = (me - s) % N_DEV
            ia_r = (me - s - 1) % N_DEV
            ib_s = (me + s) % N_DEV
            ib_r = (me + s + 1) % N_DEV
            if s >= 2:
                pl.semaphore_wait(cr_a, 1)
                pl.semaphore_wait(cr_b, 1)
                last_a[slot].wait_send()
                last_b[slot].wait_send()
            rdma_a = pltpu.make_async_remote_copy(
                src_ref=out_ref.at[pl.ds(ia_s * ch, ch), 0:ha],
                dst_ref=comm_a.at[slot],
                send_sem=sa_send.at[slot], recv_sem=sa_recv.at[slot],
                device_id=(right,), device_id_type=pl.DeviceIdType.MESH,
            )
            rdma_b = pltpu.make_async_remote_copy(
                src_ref=out_ref.at[pl.ds(ib_s * ch, ch), ha:n],
                dst_ref=comm_b.at[slot],
                send_sem=sb_send.at[slot], recv_sem=sb_recv.at[slot],
                device_id=(left,), device_id_type=pl.DeviceIdType.MESH,
            )
            rdma_a.start()
            rdma_b.start()
            last_a[slot] = rdma_a
            last_b[slot] = rdma_b
            rdma_a.wait_recv()
            out_ref[pl.ds(ia_r * ch, ch), 0:ha] = (
                out_ref[pl.ds(ia_r * ch, ch), 0:ha] + comm_a[slot]
            )
            if s < N_DEV - 3:
                pl.semaphore_signal(
                    cr_a, inc=1,
                    device_id=(left,), device_id_type=pl.DeviceIdType.MESH,
                )
            rdma_b.wait_recv()
            out_ref[pl.ds(ib_r * ch, ch), ha:n] = (
                out_ref[pl.ds(ib_r * ch, ch), ha:n] + comm_b[slot]
            )
            if s < N_DEV - 3:
                pl.semaphore_signal(
                    cr_b, inc=1,
                    device_id=(right,), device_id_type=pl.DeviceIdType.MESH,
                )
        for slot in (0, 1):
            last_a[slot].wait_send()
            last_b[slot].wait_send()

        own_a = (me + 1) % N_DEV
        own_b = (me - 1) % N_DEV
        scale = sx_ref[0] * sw_ref[0]
        out_ref[pl.ds(own_a * ch, ch), 0:ha] = jnp.maximum(
            out_ref[pl.ds(own_a * ch, ch), 0:ha] * scale, 0.0
        )
        out_ref[pl.ds(own_b * ch, ch), ha:n] = jnp.maximum(
            out_ref[pl.ds(own_b * ch, ch), ha:n] * scale, 0.0
        )

        last_ga = {}
        last_gb = {}
        for s in range(N_DEV - 1):
            slot = s % 2
            ca = (me + 1 - s) % N_DEV
            cb = (me - 1 + s) % N_DEV
            if s >= 2:
                pl.semaphore_wait(cg_a, 1)
                pl.semaphore_wait(cg_b, 1)
                last_ga[slot].wait_send()
                last_gb[slot].wait_send()
            rdma_a = pltpu.make_async_remote_copy(
                src_ref=out_ref.at[pl.ds(ca * ch, ch), 0:ha],
                dst_ref=out_ref.at[pl.ds(ca * ch, ch), 0:ha],
                send_sem=ga_send.at[slot], recv_sem=ga_recv.at[slot],
                device_id=(right,), device_id_type=pl.DeviceIdType.MESH,
            )
            rdma_b = pltpu.make_async_remote_copy(
                src_ref=out_ref.at[pl.ds(cb * ch, ch), ha:n],
                dst_ref=out_ref.at[pl.ds(cb * ch, ch), ha:n],
                send_sem=gb_send.at[slot], recv_sem=gb_recv.at[slot],
                device_id=(left,), device_id_type=pl.DeviceIdType.MESH,
            )
            rdma_a.start()
            rdma_b.start()
            last_ga[slot] = rdma_a
            last_gb[slot] = rdma_b
            rdma_a.wait_recv()
            if s < N_DEV - 3:
                pl.semaphore_signal(
                    cg_a, inc=1,
                    device_id=(left,), device_id_type=pl.DeviceIdType.MESH,
                )
            rdma_b.wait_recv()
            if s < N_DEV - 3:
                pl.semaphore_signal(
                    cg_b, inc=1,
                    device_id=(right,), device_id_type=pl.DeviceIdType.MESH,
                )
        for slot in (0, 1):
            last_ga[slot].wait_send()
            last_gb[slot].wait_send()

    return pl.pallas_call(
        body,
        out_shape=jax.ShapeDtypeStruct((m, n), jnp.float32),
        in_specs=[
            pl.BlockSpec(memory_space=pltpu.VMEM),
            pl.BlockSpec(memory_space=pltpu.VMEM),
            pl.BlockSpec(memory_space=pltpu.SMEM),
            pl.BlockSpec(memory_space=pltpu.SMEM),
        ],
        out_specs=pl.BlockSpec(memory_space=pltpu.VMEM),
        scratch_shapes=[
            pltpu.VMEM((2, ch, ha), jnp.float32),
            pltpu.VMEM((2, ch, ha), jnp.float32),
            pltpu.SemaphoreType.DMA((2,)),
            pltpu.SemaphoreType.DMA((2,)),
            pltpu.SemaphoreType.DMA((2,)),
            pltpu.SemaphoreType.DMA((2,)),
            pltpu.SemaphoreType.DMA((2,)),
            pltpu.SemaphoreType.DMA((2,)),
            pltpu.SemaphoreType.DMA((2,)),
            pltpu.SemaphoreType.DMA((2,)),
            pltpu.SemaphoreType.REGULAR,
            pltpu.SemaphoreType.REGULAR,
            pltpu.SemaphoreType.REGULAR,
            pltpu.SemaphoreType.REGULAR,
        ],
        compiler_params=pltpu.CompilerParams(
            collective_id=0, vmem_limit_bytes=100 * 1024 * 1024
        ),
    )(x, w_mat, scale_x, scale_w)


# device time: 503096 ns/iter; 1.7549x vs baseline; 1.7549x over previous
import jax
import jax.numpy as jnp
from jax import lax
from jax.experimental import pallas as pl
from jax.experimental.pallas import tpu as pltpu

N_DEV = 32


def _build_tables():
    def coords(l):
        z, p = l // 8, l % 8
        return (((p + 1) // 2) % 2, p // 2, z)

    def cpos(l):
        x, y, z = coords(l)
        j = z * 4 + (y if z % 2 == 0 else 3 - y)
        return j if x == 0 else 31 - j

    kpos = [cpos(l) for l in range(N_DEV)]
    by_k = {k: l for l, k in enumerate(kpos)}
    succ = [by_k[(kpos[l] + 1) % N_DEV] for l in range(N_DEV)]
    pred = [by_k[(kpos[l] - 1) % N_DEV] for l in range(N_DEV)]
    return kpos, succ, pred


_KPOS, _SUCC, _PRED = _build_tables()


def _sel(table, idx):
    out = jnp.int32(0)
    for i, v in enumerate(table):
        out = out + jnp.int32(v) * (idx == i).astype(jnp.int32)
    return out


def kernel(x, w_mat, scale_x, scale_w):
    m, k = x.shape
    _, n = w_mat.shape
    ch = m // N_DEV
    ha = n // 2

    def body(x_ref, w_ref, sx_ref, sw_ref, out_ref, comm_a, comm_b,
             sa_send, sa_recv, sb_send, sb_recv,
             ga_send, ga_recv, gb_send, gb_recv,
             cr_a, cr_b, cg_a, cg_b):
        me = lax.axis_index("i")
        kpos = _sel(_KPOS, me)
        right = _sel(_SUCC, me)
        left = _sel(_PRED, me)

        out_ref[...] = jnp.dot(
            x_ref[...].astype(jnp.bfloat16),
            w_ref[...].astype(jnp.bfloat16),
            preferred_element_type=jnp.float32,
        )

        barrier_sem = pltpu.get_barrier_semaphore()
        for nbr in (left, right):
            pl.semaphore_signal(
                barrier_sem, inc=1,
                device_id=(nbr,), device_id_type=pl.DeviceIdType.MESH,
            )
        pl.semaphore_wait(barrier_sem, 2)

        last_a = {}
        last_b = {}
        for s in range(N_DEV - 1):
            slot = s % 2
            ia_s = (kpos - s) % N_DEV
            ia_r = (kpos - s - 1) % N_DEV
            ib_s = (kpos + s) % N_DEV
            ib_r = (kpos + s + 1) % N_DEV
            if s >= 2:
                pl.semaphore_wait(cr_a, 1)
                pl.semaphore_wait(cr_b, 1)
                last_a[slot].wait_send()
                last_b[slot].wait_send()
            rdma_a = pltpu.make_async_remote_copy(
                src_ref=out_ref.at[pl.ds(ia_s * ch, ch), 0:ha],
                dst_ref=comm_a.at[slot],
                send_sem=sa_send.at[slot], recv_sem=sa_recv.at[slot],
                device_id=(right,), device_id_type=pl.DeviceIdType.MESH,
            )
            rdma_b = pltpu.make_async_remote_copy(
                src_ref=out_ref.at[pl.ds(ib_s * ch, ch), ha:n],
                dst_ref=comm_b.at[slot],
                send_sem=sb_send.at[slot], recv_sem=sb_recv.at[slot],
                device_id=(left,), device_id_type=pl.DeviceIdType.MESH,
            )
            rdma_a.start()
            rdma_b.start()
            last_a[slot] = rdma_a
            last_b[slot] = rdma_b
            rdma_a.wait_recv()
            out_ref[pl.ds(ia_r * ch, ch), 0:ha] = (
                out_ref[pl.ds(ia_r * ch, ch), 0:ha] + comm_a[slot]
            )
            if s < N_DEV - 3:
                pl.semaphore_signal(
                    cr_a, inc=1,
                    device_id=(left,), device_id_type=pl.DeviceIdType.MESH,
                )
            rdma_b.wait_recv()
            out_ref[pl.ds(ib_r * ch, ch), ha:n] = (
                out_ref[pl.ds(ib_r * ch, ch), ha:n] + comm_b[slot]
            )
            if s < N_DEV - 3:
                pl.semaphore_signal(
                    cr_b, inc=1,
                    device_id=(right,), device_id_type=pl.DeviceIdType.MESH,
                )
        for slot in (0, 1):
            last_a[slot].wait_send()
            last_b[slot].wait_send()

        own_a = (kpos + 1) % N_DEV
        own_b = (kpos - 1) % N_DEV
        scale = sx_ref[0] * sw_ref[0]
        out_ref[pl.ds(own_a * ch, ch), 0:ha] = jnp.maximum(
            out_ref[pl.ds(own_a * ch, ch), 0:ha] * scale, 0.0
        )
        out_ref[pl.ds(own_b * ch, ch), ha:n] = jnp.maximum(
            out_ref[pl.ds(own_b * ch, ch), ha:n] * scale, 0.0
        )

        last_ga = {}
        last_gb = {}
        for s in range(N_DEV - 1):
            slot = s % 2
            ca = (kpos + 1 - s) % N_DEV
            cb = (kpos - 1 + s) % N_DEV
            if s >= 2:
                pl.semaphore_wait(cg_a, 1)
                pl.semaphore_wait(cg_b, 1)
                last_ga[slot].wait_send()
                last_gb[slot].wait_send()
            rdma_a = pltpu.make_async_remote_copy(
                src_ref=out_ref.at[pl.ds(ca * ch, ch), 0:ha],
                dst_ref=out_ref.at[pl.ds(ca * ch, ch), 0:ha],
                send_sem=ga_send.at[slot], recv_sem=ga_recv.at[slot],
                device_id=(right,), device_id_type=pl.DeviceIdType.MESH,
            )
            rdma_b = pltpu.make_async_remote_copy(
                src_ref=out_ref.at[pl.ds(cb * ch, ch), ha:n],
                dst_ref=out_ref.at[pl.ds(cb * ch, ch), ha:n],
                send_sem=gb_send.at[slot], recv_sem=gb_recv.at[slot],
                device_id=(left,), device_id_type=pl.DeviceIdType.MESH,
            )
            rdma_a.start()
            rdma_b.start()
            last_ga[slot] = rdma_a
            last_gb[slot] = rdma_b
            rdma_a.wait_recv()
            if s < N_DEV - 3:
                pl.semaphore_signal(
                    cg_a, inc=1,
                    device_id=(left,), device_id_type=pl.DeviceIdType.MESH,
                )
            rdma_b.wait_recv()
            if s < N_DEV - 3:
                pl.semaphore_signal(
                    cg_b, inc=1,
                    device_id=(right,), device_id_type=pl.DeviceIdType.MESH,
                )
        for slot in (0, 1):
            last_ga[slot].wait_send()
            last_gb[slot].wait_send()

    return pl.pallas_call(
        body,
        out_shape=jax.ShapeDtypeStruct((m, n), jnp.float32),
        in_specs=[
            pl.BlockSpec(memory_space=pltpu.VMEM),
            pl.BlockSpec(memory_space=pltpu.VMEM),
            pl.BlockSpec(memory_space=pltpu.SMEM),
            pl.BlockSpec(memory_space=pltpu.SMEM),
        ],
        out_specs=pl.BlockSpec(memory_space=pltpu.VMEM),
        scratch_shapes=[
            pltpu.VMEM((2, ch, ha), jnp.float32),
            pltpu.VMEM((2, ch, ha), jnp.float32),
            pltpu.SemaphoreType.DMA((2,)),
            pltpu.SemaphoreType.DMA((2,)),
            pltpu.SemaphoreType.DMA((2,)),
            pltpu.SemaphoreType.DMA((2,)),
            pltpu.SemaphoreType.DMA((2,)),
            pltpu.SemaphoreType.DMA((2,)),
            pltpu.SemaphoreType.DMA((2,)),
            pltpu.SemaphoreType.DMA((2,)),
            pltpu.SemaphoreType.REGULAR,
            pltpu.SemaphoreType.REGULAR,
            pltpu.SemaphoreType.REGULAR,
            pltpu.SemaphoreType.REGULAR,
        ],
        compiler_params=pltpu.CompilerParams(
            collective_id=0, vmem_limit_bytes=100 * 1024 * 1024
        ),
    )(x, w_mat, scale_x, scale_w)


# device time: 500644 ns/iter; 1.7635x vs baseline; 1.0049x over previous
import jax
import jax.numpy as jnp
from jax import lax
from jax.experimental import pallas as pl
from jax.experimental.pallas import tpu as pltpu

N_DEV = 32
N_STEP = N_DEV - 1


def _build_tables():
    def coords(l):
        z, p = l // 8, l % 8
        return (((p + 1) // 2) % 2, p // 2, z)

    def cpos(l):
        x, y, z = coords(l)
        j = z * 4 + (y if z % 2 == 0 else 3 - y)
        return j if x == 0 else 31 - j

    kpos = [cpos(l) for l in range(N_DEV)]
    by_k = {k: l for l, k in enumerate(kpos)}
    succ = [by_k[(kpos[l] + 1) % N_DEV] for l in range(N_DEV)]
    pred = [by_k[(kpos[l] - 1) % N_DEV] for l in range(N_DEV)]
    return kpos, succ, pred


_KPOS, _SUCC, _PRED = _build_tables()


def _sel(table, idx):
    out = jnp.int32(0)
    for i, v in enumerate(table):
        out = out + jnp.int32(v) * (idx == i).astype(jnp.int32)
    return out


def kernel(x, w_mat, scale_x, scale_w):
    m, k = x.shape
    _, n = w_mat.shape
    ch = m // N_DEV
    ha = n // 2

    def body(x_ref, w_ref, sx_ref, sw_ref, out_ref, comm_a, comm_b,
             sa_send, sa_recv, sb_send, sb_recv,
             ga_send, ga_recv, gb_send, gb_recv,
             cr_a, cr_b, cg_a, cg_b):
        me = lax.axis_index("i")
        kpos = _sel(_KPOS, me)
        right = _sel(_SUCC, me)
        left = _sel(_PRED, me)

        out_ref[...] = jnp.dot(
            x_ref[...].astype(jnp.bfloat16),
            w_ref[...].astype(jnp.bfloat16),
            preferred_element_type=jnp.float32,
        )

        barrier_sem = pltpu.get_barrier_semaphore()
        for nbr in (left, right):
            pl.semaphore_signal(
                barrier_sem, inc=1,
                device_id=(nbr,), device_id_type=pl.DeviceIdType.MESH,
            )
        pl.semaphore_wait(barrier_sem, 2)

        def rs_send(ring, s):
            slot = s % 2
            if ring == 0:
                c = (kpos - s) % N_DEV
                return pltpu.make_async_remote_copy(
                    src_ref=out_ref.at[pl.ds(c * ch, ch), 0:ha],
                    dst_ref=comm_a.at[slot],
                    send_sem=sa_send.at[slot], recv_sem=sa_recv.at[slot],
                    device_id=(right,),
                    device_id_type=pl.DeviceIdType.MESH,
                )
            c = (kpos + s) % N_DEV
            return pltpu.make_async_remote_copy(
                src_ref=out_ref.at[pl.ds(c * ch, ch), ha:n],
                dst_ref=comm_b.at[slot],
                send_sem=sb_send.at[slot], recv_sem=sb_recv.at[slot],
                device_id=(left,),
                device_id_type=pl.DeviceIdType.MESH,
            )

        rda = [None] * N_STEP
        rdb = [None] * N_STEP
        rda[0] = rs_send(0, 0)
        rdb[0] = rs_send(1, 0)
        rda[0].start()
        rdb[0].start()
        for s in range(N_STEP):
            ia_r = (kpos - s - 1) % N_DEV
            ib_r = (kpos + s + 1) % N_DEV
            rda[s].wait_recv()
            out_ref[pl.ds(ia_r * ch, ch), 0:ha] = (
                out_ref[pl.ds(ia_r * ch, ch), 0:ha] + comm_a[s % 2]
            )
            if s < N_STEP - 2:
                pl.semaphore_signal(
                    cr_a, inc=1,
                    device_id=(left,), device_id_type=pl.DeviceIdType.MESH,
                )
            if s < N_STEP - 1:
                if s + 1 >= 2:
                    pl.semaphore_wait(cr_a, 1)
                    rda[s - 1].wait_send()
                rda[s + 1] = rs_send(0, s + 1)
                rda[s + 1].start()
            rdb[s].wait_recv()
            out_ref[pl.ds(ib_r * ch, ch), ha:n] = (
                out_ref[pl.ds(ib_r * ch, ch), ha:n] + comm_b[s % 2]
            )
            if s < N_STEP - 2:
                pl.semaphore_signal(
                    cr_b, inc=1,
                    device_id=(right,), device_id_type=pl.DeviceIdType.MESH,
                )
            if s < N_STEP - 1:
                if s + 1 >= 2:
                    pl.semaphore_wait(cr_b, 1)
                    rdb[s - 1].wait_send()
                rdb[s + 1] = rs_send(1, s + 1)
                rdb[s + 1].start()
        for rd in (rda, rdb):
            rd[N_STEP - 2].wait_send()
            rd[N_STEP - 1].wait_send()

        own_a = (kpos + 1) % N_DEV
        own_b = (kpos - 1) % N_DEV
        scale = sx_ref[0] * sw_ref[0]
        out_ref[pl.ds(own_a * ch, ch), 0:ha] = jnp.maximum(
            out_ref[pl.ds(own_a * ch, ch), 0:ha] * scale, 0.0
        )
        out_ref[pl.ds(own_b * ch, ch), ha:n] = jnp.maximum(
            out_ref[pl.ds(own_b * ch, ch), ha:n] * scale, 0.0
        )

        def ag_send(ring, s):
            slot = s % 2
            if ring == 0:
                c = (kpos + 1 - s) % N_DEV
                sl = out_ref.at[pl.ds(c * ch, ch), 0:ha]
                return pltpu.make_async_remote_copy(
                    src_ref=sl, dst_ref=sl,
                    send_sem=ga_send.at[slot], recv_sem=ga_recv.at[slot],
                    device_id=(right,),
                    device_id_type=pl.DeviceIdType.MESH,
                )
            c = (kpos - 1 + s) % N_DEV
            sl = out_ref.at[pl.ds(c * ch, ch), ha:n]
            return pltpu.make_async_remote_copy(
                src_ref=sl, dst_ref=sl,
                send_sem=gb_send.at[slot], recv_sem=gb_recv.at[slot],
                device_id=(left,),
                device_id_type=pl.DeviceIdType.MESH,
            )

        gda = [None] * N_STEP
        gdb = [None] * N_STEP
        gda[0] = ag_send(0, 0)
        gdb[0] = ag_send(1, 0)
        gda[0].start()
        gdb[0].start()
        for s in range(N_STEP):
            gda[s].wait_recv()
            if s < N_STEP - 2:
                pl.semaphore_signal(
                    cg_a, inc=1,
                    device_id=(left,), device_id_type=pl.DeviceIdType.MESH,
                )
            if s < N_STEP - 1:
                if s + 1 >= 2:
                    pl.semaphore_wait(cg_a, 1)
                    gda[s - 1].wait_send()
                gda[s + 1] = ag_send(0, s + 1)
                gda[s + 1].start()
            gdb[s].wait_recv()
            if s < N_STEP - 2:
                pl.semaphore_signal(
                    cg_b, inc=1,
                    device_id=(right,), device_id_type=pl.DeviceIdType.MESH,
                )
            if s < N_STEP - 1:
                if s + 1 >= 2:
                    pl.semaphore_wait(cg_b, 1)
                    gdb[s - 1].wait_send()
                gdb[s + 1] = ag_send(1, s + 1)
                gdb[s + 1].start()
        for gd in (gda, gdb):
            gd[N_STEP - 2].wait_send()
            gd[N_STEP - 1].wait_send()

    return pl.pallas_call(
        body,
        out_shape=jax.ShapeDtypeStruct((m, n), jnp.float32),
        in_specs=[
            pl.BlockSpec(memory_space=pltpu.VMEM),
            pl.BlockSpec(memory_space=pltpu.VMEM),
            pl.BlockSpec(memory_space=pltpu.SMEM),
            pl.BlockSpec(memory_space=pltpu.SMEM),
        ],
        out_specs=pl.BlockSpec(memory_space=pltpu.VMEM),
        scratch_shapes=[
            pltpu.VMEM((2, ch, ha), jnp.float32),
            pltpu.VMEM((2, ch, ha), jnp.float32),
            pltpu.SemaphoreType.DMA((2,)),
            pltpu.SemaphoreType.DMA((2,)),
            pltpu.SemaphoreType.DMA((2,)),
            pltpu.SemaphoreType.DMA((2,)),
            pltpu.SemaphoreType.DMA((2,)),
            pltpu.SemaphoreType.DMA((2,)),
            pltpu.SemaphoreType.DMA((2,)),
            pltpu.SemaphoreType.DMA((2,)),
            pltpu.SemaphoreType.REGULAR,
            pltpu.SemaphoreType.REGULAR,
            pltpu.SemaphoreType.REGULAR,
            pltpu.SemaphoreType.REGULAR,
        ],
        compiler_params=pltpu.CompilerParams(
            collective_id=0, vmem_limit_bytes=100 * 1024 * 1024
        ),
    )(x, w_mat, scale_x, scale_w)
